# baseline (speedup 1.0000x reference)
import functools

import jax
import jax.numpy as jnp
from jax import lax
from jax.experimental import pallas as pl
from jax.experimental.pallas import tpu as pltpu
from jax.experimental.pallas import tpu_sc as plsc

_NUM_EMB = 8192
_DIM = 256
_TOKENS = 8 * 32 * 32
_BT = 1024
_BK = 4096


def _dist_argmin_body(colf_ref, z_ref, cb_ref, idx_ref,
                      run_min, run_idx, zn_all, cn_cur):
    j = pl.program_id(0)
    i = pl.program_id(1)
    row = pl.ds(i * _BT, _BT)

    zb = z_ref[...]
    cb = cb_ref[...]

    @pl.when(j == 0)
    def _zn():
        run_min[row, :] = jnp.full((_BT, 1), jnp.inf, jnp.float32)
        zn_all[row, :] = jnp.sum(zb * zb, axis=1, keepdims=True)

    @pl.when(i == 0)
    def _cn():
        cn_cur[...] = jnp.sum(cb * cb, axis=1)[None, :]

    mm = lax.dot_general(
        zb * jnp.float32(-2.0), cb,
        dimension_numbers=(((1,), (1,)), ((), ())),
        preferred_element_type=jnp.float32,
        precision=lax.Precision.DEFAULT)
    s = (zn_all[row, :] + mm) + cn_cur[...]
    vmin = jnp.min(s, axis=1, keepdims=True)
    amin = jnp.min(jnp.where(s == vmin, colf_ref[...], jnp.inf),
                   axis=1, keepdims=True)
    better = vmin < run_min[row, :]
    run_idx[row, :] = jnp.where(better, amin, run_idx[row, :])
    run_min[row, :] = jnp.where(better, vmin, run_min[row, :])

    @pl.when(j == pl.num_programs(0) - 1)
    def _flush():
        idx_ref[...] = run_idx[row, :].astype(jnp.int32)


_dist_argmin = pl.pallas_call(
    _dist_argmin_body,
    grid=(_NUM_EMB // _BK, _TOKENS // _BT),
    in_specs=[
        pl.BlockSpec((1, _BK), lambda j, i: (0, j)),
        pl.BlockSpec((_BT, _DIM), lambda j, i: (i, 0)),
        pl.BlockSpec((_BK, _DIM), lambda j, i: (j, 0)),
    ],
    out_specs=pl.BlockSpec((_BT, 1), lambda j, i: (i, 0)),
    out_shape=jax.ShapeDtypeStruct((_TOKENS, 1), jnp.int32),
    scratch_shapes=[
        pltpu.VMEM((_TOKENS, 1), jnp.float32),
        pltpu.VMEM((_TOKENS, 1), jnp.float32),
        pltpu.VMEM((_TOKENS, 1), jnp.float32),
        pltpu.VMEM((1, _BK), jnp.float32),
    ],
)


def _make_sc_gather():
    info = plsc.get_sparse_core_info()
    nw = info.num_cores * info.num_subcores
    b_per_w = _TOKENS // nw
    mesh = plsc.VectorSubcoreMesh(core_axis_name="c", subcore_axis_name="s")

    @functools.partial(
        pl.kernel, mesh=mesh,
        out_type=jax.ShapeDtypeStruct((_TOKENS, _DIM), jnp.float32),
        scratch_types=[
            pltpu.VMEM((b_per_w,), jnp.int32),
            pltpu.VMEM((b_per_w, _DIM), jnp.float32),
            pltpu.SemaphoreType.DMA,
        ],
    )
    def gather(cb_hbm, idx_hbm, out_hbm, idx_v, rows_v, sem):
        wid = lax.axis_index("s") * info.num_cores + lax.axis_index("c")
        base = wid * b_per_w
        pltpu.sync_copy(idx_hbm.at[pl.ds(base, b_per_w)], idx_v)
        pltpu.async_copy(cb_hbm.at[idx_v], rows_v, sem).wait()
        pltpu.sync_copy(rows_v, out_hbm.at[pl.ds(base, b_per_w)])

    return gather


_sc_gather_cache = []


def _sc_gather(codebook, indices):
    if not _sc_gather_cache:
        _sc_gather_cache.append(_make_sc_gather())
    return _sc_gather_cache[0](codebook, indices)


def _st_loss_body(z_ref, zq_ref, st_ref, loss_ref):
    z = z_ref[...]
    d = zq_ref[...] - z
    st_ref[...] = z + d
    mean_sq = jnp.sum(d * d) / jnp.float32(_TOKENS * _DIM)
    loss_ref[...] = jnp.reshape(mean_sq + jnp.float32(0.25) * mean_sq, (1, 1))


_st_loss = pl.pallas_call(
    _st_loss_body,
    out_shape=(
        jax.ShapeDtypeStruct((_TOKENS, _DIM), jnp.float32),
        jax.ShapeDtypeStruct((1, 1), jnp.float32),
    ),
)


def kernel(z_e, codebook):
    z = jnp.transpose(z_e, (0, 2, 3, 1))
    z_flat = z.reshape(-1, _DIM)
    colf = jnp.arange(_NUM_EMB, dtype=jnp.float32).reshape(1, _NUM_EMB)
    idx2 = _dist_argmin(colf, z_flat, codebook)
    indices = idx2.reshape(-1)
    zq_flat = _sc_gather(codebook, indices)
    st_flat, loss11 = _st_loss(z_flat, zq_flat)
    zq_out = jnp.transpose(st_flat.reshape(z.shape), (0, 3, 1, 2))
    return (zq_out, loss11.reshape(()), indices)


# speedup vs baseline: 1.0745x; 1.0745x over previous
import functools

import jax
import jax.numpy as jnp
from jax import lax
from jax.experimental import pallas as pl
from jax.experimental.pallas import tpu as pltpu
from jax.experimental.pallas import tpu_sc as plsc

_NUM_EMB = 8192
_DIM = 256
_TOKENS = 8 * 32 * 32
_BT = 1024
_BK = 4096


def _dist_argmin_body(colf_ref, z_ref, cb_ref, idx_ref,
                      run_min, run_idx):
    j = pl.program_id(1)

    @pl.when(j == 0)
    def _init():
        run_min[...] = jnp.full(run_min.shape, jnp.inf, run_min.dtype)
        run_idx[...] = jnp.zeros(run_idx.shape, run_idx.dtype)

    zb = z_ref[...]
    cb = cb_ref[...]
    zn = jnp.sum(zb * zb, axis=1, keepdims=True)
    cn = jnp.sum(cb * cb, axis=1)[None, :]
    mm = lax.dot_general(
        zb * jnp.float32(-2.0), cb,
        dimension_numbers=(((1,), (1,)), ((), ())),
        preferred_element_type=jnp.float32,
        precision=lax.Precision.DEFAULT)
    s = (zn + mm) + cn
    vmin = jnp.min(s, axis=1, keepdims=True)
    amin = jnp.min(jnp.where(s == vmin, colf_ref[...], jnp.inf),
                   axis=1, keepdims=True)
    better = vmin < run_min[...]
    run_idx[...] = jnp.where(better, amin, run_idx[...])
    run_min[...] = jnp.where(better, vmin, run_min[...])

    @pl.when(j == pl.num_programs(1) - 1)
    def _flush():
        idx_ref[...] = run_idx[...].astype(jnp.int32)


_dist_argmin = pl.pallas_call(
    _dist_argmin_body,
    grid=(_TOKENS // _BT, _NUM_EMB // _BK),
    in_specs=[
        pl.BlockSpec((1, _BK), lambda i, j: (0, j)),
        pl.BlockSpec((_BT, _DIM), lambda i, j: (i, 0)),
        pl.BlockSpec((_BK, _DIM), lambda i, j: (j, 0)),
    ],
    out_specs=pl.BlockSpec((_BT, 1), lambda i, j: (i, 0)),
    out_shape=jax.ShapeDtypeStruct((_TOKENS, 1), jnp.int32),
    scratch_shapes=[
        pltpu.VMEM((_BT, 1), jnp.float32),
        pltpu.VMEM((_BT, 1), jnp.float32),
    ],
)


def _make_sc_gather():
    info = plsc.get_sparse_core_info()
    nw = info.num_cores * info.num_subcores
    b_per_w = _TOKENS // nw
    mesh = plsc.VectorSubcoreMesh(core_axis_name="c", subcore_axis_name="s")

    @functools.partial(
        pl.kernel, mesh=mesh,
        out_type=jax.ShapeDtypeStruct((_TOKENS, _DIM), jnp.float32),
        scratch_types=[
            pltpu.VMEM((b_per_w,), jnp.int32),
            pltpu.VMEM((b_per_w, _DIM), jnp.float32),
            pltpu.SemaphoreType.DMA,
        ],
    )
    def gather(cb_hbm, idx_hbm, out_hbm, idx_v, rows_v, sem):
        wid = lax.axis_index("s") * info.num_cores + lax.axis_index("c")
        base = wid * b_per_w
        pltpu.sync_copy(idx_hbm.at[pl.ds(base, b_per_w)], idx_v)
        pltpu.async_copy(cb_hbm.at[idx_v], rows_v, sem).wait()
        pltpu.sync_copy(rows_v, out_hbm.at[pl.ds(base, b_per_w)])

    return gather


_sc_gather_cache = []


def _sc_gather(codebook, indices):
    if not _sc_gather_cache:
        _sc_gather_cache.append(_make_sc_gather())
    return _sc_gather_cache[0](codebook, indices)


_BL = 1024


def _loss_body(z_ref, zq_ref, loss_ref, acc_ref):
    i = pl.program_id(0)
    d = zq_ref[...] - z_ref[...]
    part = jnp.sum(d * d)

    @pl.when(i == 0)
    def _init():
        acc_ref[0] = part

    @pl.when(i > 0)
    def _acc():
        acc_ref[0] = acc_ref[0] + part

    @pl.when(i == pl.num_programs(0) - 1)
    def _flush():
        mean_sq = acc_ref[0] / jnp.float32(_TOKENS * _DIM)
        loss_ref[...] = jnp.reshape(mean_sq + jnp.float32(0.25) * mean_sq,
                                    (1, 1))


_loss = pl.pallas_call(
    _loss_body,
    grid=(_TOKENS // _BL,),
    in_specs=[
        pl.BlockSpec((_BL, _DIM), lambda i: (i, 0)),
        pl.BlockSpec((_BL, _DIM), lambda i: (i, 0)),
    ],
    out_specs=pl.BlockSpec((1, 1), lambda i: (0, 0)),
    out_shape=jax.ShapeDtypeStruct((1, 1), jnp.float32),
    scratch_shapes=[pltpu.SMEM((1,), jnp.float32)],
)


def kernel(z_e, codebook):
    z = jnp.transpose(z_e, (0, 2, 3, 1))
    z_flat = z.reshape(-1, _DIM)
    colf = jnp.arange(_NUM_EMB, dtype=jnp.float32).reshape(1, _NUM_EMB)
    idx2 = _dist_argmin(colf, z_flat, codebook)
    indices = idx2.reshape(-1)
    zq_flat = _sc_gather(codebook, indices)
    loss11 = _loss(z_flat, zq_flat)
    zq_out = jnp.transpose(zq_flat.reshape(z.shape), (0, 3, 1, 2))
    return (zq_out, loss11.reshape(()), indices)
